# Initial kernel scaffold; baseline (speedup 1.0000x reference)
import functools

import jax
import jax.numpy as jnp
from jax import lax
from jax.experimental import pallas as pl
from jax.experimental.pallas import tpu as pltpu
from jax.experimental.pallas import tpu_sc as plsc

G = 128
NS = 16
NC = 2
STEPS = 2


def _sc_agg_kernel(seq, n, e, d):
    ng = e // G
    base_g, extra = divmod(ng, NS)
    rows_per_tile = n // NS
    zrows = 125
    assert rows_per_tile % zrows == 0
    mesh = plsc.VectorSubcoreMesh(core_axis_name="c", subcore_axis_name="s")

    @functools.partial(
        pl.kernel,
        out_type=[
            jax.ShapeDtypeStruct((seq * n, d), jnp.float32),
            jax.ShapeDtypeStruct((n, 16), jnp.float32),
        ],
        mesh=mesh,
        scratch_types=[
            pltpu.VMEM((G, d), jnp.float32),
            pltpu.VMEM((G,), jnp.int32),
            pltpu.VMEM((1, G), jnp.int32),
            pltpu.VMEM((G, 16), jnp.float32),
            pltpu.VMEM((125, d), jnp.float32),
            pltpu.VMEM((n // NS, 16), jnp.float32),
            pltpu.VMEM_SHARED((n, d), jnp.float32),
            pltpu.VMEM_SHARED((n, 16), jnp.float32),
            pltpu.SemaphoreType.DMA,
        ],
    )
    def kern(x_hbm, src_hbm, dst_hbm, out_hbm, deg_hbm,
             rows_v, sidx_v, didx_v, ones_v, zer_v, zdeg_v, agg_s, deg_s, sem):
        c = lax.axis_index("c")
        s = lax.axis_index("s")

        zero16 = jnp.zeros((16,), jnp.float32)
        one16 = jnp.ones((16,), jnp.float32)

        def init_zer(i, _):
            for j in range(d // 16):
                zer_v[i, pl.ds(j * 16, 16)] = zero16
            return 0
        lax.fori_loop(0, zrows, init_zer, 0)

        def init_zdeg(i, _):
            zdeg_v[i, :] = zero16
            return 0
        lax.fori_loop(0, n // NS, init_zdeg, 0)

        def init_ones(i, _):
            ones_v[i, :] = one16
            return 0
        lax.fori_loop(0, G, init_ones, 0)

        r0 = s * rows_per_tile
        g0 = s * base_g + jnp.minimum(s, extra)
        cnt = base_g + jnp.where(s < extra, 1, 0)

        for step in range(STEPS):
            t = STEPS * c + step

            for j in range(rows_per_tile // zrows):
                pltpu.sync_copy(zer_v, agg_s.at[pl.ds(r0 + j * zrows, zrows)])
            if step == 0:
                @pl.when(c == 0)
                def _():
                    pltpu.sync_copy(zdeg_v, deg_s.at[pl.ds(r0, rows_per_tile)])
            plsc.subcore_barrier()

            def group_body(g, _):
                gg = g0 + g
                pltpu.sync_copy(src_hbm.at[t * ng + gg], sidx_v)
                pltpu.sync_copy(dst_hbm.at[gg], didx_v.at[0])
                pltpu.async_copy(x_hbm.at[sidx_v], rows_v, sem).wait()
                pltpu.sync_copy(rows_v, agg_s.at[didx_v.at[0]], add=True)
                if step == 0:
                    @pl.when(c == 0)
                    def _():
                        pltpu.sync_copy(ones_v, deg_s.at[didx_v.at[0]],
                                        add=True)
                return 0
            lax.fori_loop(0, cnt, group_body, 0)
            plsc.subcore_barrier()

            for j in range(rows_per_tile // zrows):
                rr = r0 + j * zrows
                pltpu.sync_copy(agg_s.at[pl.ds(rr, zrows)],
                                out_hbm.at[pl.ds(t * n + rr, zrows)])
            if step == 0:
                @pl.when(c == 0)
                def _():
                    pltpu.sync_copy(deg_s.at[pl.ds(r0, rows_per_tile)],
                                    deg_hbm.at[pl.ds(r0, rows_per_tile)])

    return kern


def _tc_body(deg_ref, x_ref, agg_ref, wl_ref, wr_ref, bl_ref, fcw_ref,
             fcb_ref, h_ref, y_ref):
    inv = 1.0 / jnp.maximum(deg_ref[:, 0:1], 1.0)
    a = agg_ref[0] * inv
    h = (jnp.dot(a, wl_ref[...], preferred_element_type=jnp.float32,
                 precision=lax.Precision.HIGHEST)
         + jnp.dot(x_ref[0], wr_ref[...], preferred_element_type=jnp.float32,
                   precision=lax.Precision.HIGHEST)
         + bl_ref[...])
    h_ref[0] = h
    y_ref[0] = jnp.sum(h * fcw_ref[...], axis=1, keepdims=True) + fcb_ref[0]


def kernel(input, edge_index, W_l, b_l, W_r, fc_w, fc_b):
    seq, n, d = input.shape
    e = edge_index.shape[1]
    assert e % G == 0
    ng = e // G

    x_flat = input.reshape(seq * n, d)
    src4 = (edge_index[0][None, :]
            + (jnp.arange(seq, dtype=jnp.int32) * n)[:, None])
    src_g = src4.reshape(seq * ng, G)
    dst_g = edge_index[1].reshape(ng, G)

    agg_flat, deg = _sc_agg_kernel(seq, n, e, d)(x_flat, src_g, dst_g)
    agg3 = agg_flat.reshape(seq, n, d)

    blk = 2000
    h3, y3 = pl.pallas_call(
        _tc_body,
        grid=(seq, n // blk),
        in_specs=[
            pl.BlockSpec((blk, 16), lambda t, b: (b, 0)),
            pl.BlockSpec((1, blk, d), lambda t, b: (t, b, 0)),
            pl.BlockSpec((1, blk, d), lambda t, b: (t, b, 0)),
            pl.BlockSpec((d, d), lambda t, b: (0, 0)),
            pl.BlockSpec((d, d), lambda t, b: (0, 0)),
            pl.BlockSpec((1, d), lambda t, b: (0, 0)),
            pl.BlockSpec((1, d), lambda t, b: (0, 0)),
            pl.BlockSpec(memory_space=pltpu.SMEM),
        ],
        out_specs=[
            pl.BlockSpec((1, blk, d), lambda t, b: (t, b, 0)),
            pl.BlockSpec((1, blk, 1), lambda t, b: (t, b, 0)),
        ],
        out_shape=[
            jax.ShapeDtypeStruct((seq, n, d), jnp.float32),
            jax.ShapeDtypeStruct((seq, n, 1), jnp.float32),
        ],
    )(deg, input, agg3, W_l, W_r, b_l.reshape(1, d), fc_w.T, fc_b)

    return h3, y3[..., 0]


# speedup vs baseline: 3.3006x; 3.3006x over previous
import functools

import jax
import jax.numpy as jnp
from jax import lax
from jax.experimental import pallas as pl
from jax.experimental.pallas import tpu as pltpu
from jax.experimental.pallas import tpu_sc as plsc

G = 128
GB = 8
NS = 16
NC = 2
NA = 10240
ZR = 128


def _sc_agg_kernel(seq, n, ep, d):
    dh = d // NC
    nblk = ep // (G * GB)
    bpt = nblk // NS
    rpt = NA // NS
    mesh = plsc.VectorSubcoreMesh(core_axis_name="c", subcore_axis_name="s")

    @functools.partial(
        pl.kernel,
        out_type=[
            jax.ShapeDtypeStruct((NC * seq * NA, dh), jnp.float32),
            jax.ShapeDtypeStruct((NA, 16), jnp.float32),
        ],
        mesh=mesh,
        compiler_params=pltpu.CompilerParams(use_tc_tiling_on_sc=False),
        scratch_types=[
            pltpu.VMEM((G, dh), jnp.float32),
            pltpu.VMEM((GB, G), jnp.int32),
            pltpu.VMEM((GB, G), jnp.int32),
            pltpu.VMEM((G, 16), jnp.float32),
            pltpu.VMEM((ZR, dh), jnp.float32),
            pltpu.VMEM((rpt, 16), jnp.float32),
            pltpu.VMEM_SHARED((NA, dh), jnp.float32),
            pltpu.VMEM_SHARED((NA, 16), jnp.float32),
            pltpu.SemaphoreType.DMA,
        ],
    )
    def kern(x_hbm, src_hbm, dst_hbm, out_hbm, deg_hbm,
             rows_v, sidx_v, didx_v, ones_v, zer_v, zdeg_v, agg_s, deg_s, sem):
        c = lax.axis_index("c")
        s = lax.axis_index("s")

        zero16 = jnp.zeros((16,), jnp.float32)
        one16 = jnp.ones((16,), jnp.float32)
        ione16 = jnp.ones((16,), jnp.int32)

        def init_zer(i, _):
            for j in range(dh // 16):
                zer_v[i, pl.ds(j * 16, 16)] = zero16
            return 0
        lax.fori_loop(0, ZR, init_zer, 0)

        def init_zdeg(i, _):
            zdeg_v[i, :] = zero16
            return 0
        lax.fori_loop(0, rpt, init_zdeg, 0)

        def init_ones(i, _):
            ones_v[i, :] = one16
            return 0
        lax.fori_loop(0, G, init_ones, 0)

        r0 = s * rpt
        b0 = s * bpt

        for t in range(seq):
            off_v = ((c * seq + t) * n) * ione16

            for j in range(rpt // ZR):
                pltpu.sync_copy(zer_v, agg_s.at[pl.ds(r0 + j * ZR, ZR)])
            if t == 0:
                @pl.when(c == 0)
                def _():
                    pltpu.sync_copy(zdeg_v, deg_s.at[pl.ds(r0, rpt)])
            plsc.subcore_barrier()

            def block_body(b, _):
                pltpu.sync_copy(src_hbm.at[b0 + b], sidx_v)
                pltpu.sync_copy(dst_hbm.at[b0 + b], didx_v)
                for j in range(GB):
                    for k in range(G // 16):
                        sl = pl.ds(k * 16, 16)
                        sidx_v[j, sl] = sidx_v[j, sl] + off_v
                for j in range(GB):
                    pltpu.async_copy(x_hbm.at[sidx_v.at[j]], rows_v,
                                     sem).wait()
                    pltpu.sync_copy(rows_v, agg_s.at[didx_v.at[j]], add=True)
                    if t == 0:
                        @pl.when(c == 0)
                        def _():
                            pltpu.sync_copy(ones_v, deg_s.at[didx_v.at[j]],
                                            add=True)
                return 0
            lax.fori_loop(0, bpt, block_body, 0)
            plsc.subcore_barrier()

            for j in range(rpt // ZR):
                rr = r0 + j * ZR
                pltpu.sync_copy(agg_s.at[pl.ds(rr, ZR)],
                                out_hbm.at[pl.ds((c * seq + t) * NA + rr, ZR)])
            if t == 0:
                @pl.when(c == 0)
                def _():
                    pltpu.sync_copy(deg_s.at[pl.ds(r0, rpt)],
                                    deg_hbm.at[pl.ds(r0, rpt)])

    return kern


def _tc_body(deg_ref, x_ref, alo_ref, ahi_ref, wll_ref, wlh_ref, wr_ref,
             bl_ref, fcw_ref, fcb_ref, h_ref, y_ref):
    inv = 1.0 / jnp.maximum(deg_ref[:, 0:1], 1.0)
    hp = lax.Precision.HIGHEST
    f32 = jnp.float32
    h = (jnp.dot(alo_ref[0] * inv, wll_ref[...], preferred_element_type=f32,
                 precision=hp)
         + jnp.dot(ahi_ref[0] * inv, wlh_ref[...], preferred_element_type=f32,
                   precision=hp)
         + jnp.dot(x_ref[0], wr_ref[...], preferred_element_type=f32,
                   precision=hp)
         + bl_ref[...])
    h_ref[0] = h
    y_ref[0] = jnp.sum(h * fcw_ref[...], axis=1, keepdims=True) + fcb_ref[0]


def kernel(input, edge_index, W_l, b_l, W_r, fc_w, fc_b):
    seq, n, d = input.shape
    e = edge_index.shape[1]
    dh = d // NC
    epad = G * GB * NS
    ep = ((e + epad - 1) // epad) * epad
    ng = ep // G

    xc = input.reshape(seq * n, NC, dh).transpose(1, 0, 2).reshape(
        NC * seq * n, dh)
    src = edge_index[0]
    dst = edge_index[1]
    if ep != e:
        src = jnp.concatenate([src, jnp.zeros((ep - e,), src.dtype)])
        dst = jnp.concatenate([dst, jnp.full((ep - e,), n, dst.dtype)])
    src_g = src.reshape(ng // GB, GB, G)
    dst_g = dst.reshape(ng // GB, GB, G)

    agg_flat, deg = _sc_agg_kernel(seq, n, ep, d)(xc, src_g, dst_g)
    agg4 = agg_flat.reshape(NC, seq, NA, dh)

    blk = 2000
    h3, y3 = pl.pallas_call(
        _tc_body,
        grid=(seq, n // blk),
        in_specs=[
            pl.BlockSpec((blk, 16), lambda t, b: (b, 0)),
            pl.BlockSpec((1, blk, d), lambda t, b: (t, b, 0)),
            pl.BlockSpec((1, blk, dh), lambda t, b: (t, b, 0)),
            pl.BlockSpec((1, blk, dh), lambda t, b: (t, b, 0)),
            pl.BlockSpec((dh, d), lambda t, b: (0, 0)),
            pl.BlockSpec((dh, d), lambda t, b: (0, 0)),
            pl.BlockSpec((d, d), lambda t, b: (0, 0)),
            pl.BlockSpec((1, d), lambda t, b: (0, 0)),
            pl.BlockSpec((1, d), lambda t, b: (0, 0)),
            pl.BlockSpec(memory_space=pltpu.SMEM),
        ],
        out_specs=[
            pl.BlockSpec((1, blk, d), lambda t, b: (t, b, 0)),
            pl.BlockSpec((1, blk, 1), lambda t, b: (t, b, 0)),
        ],
        out_shape=[
            jax.ShapeDtypeStruct((seq, n, d), jnp.float32),
            jax.ShapeDtypeStruct((seq, n, 1), jnp.float32),
        ],
    )(deg, input, agg4[0], agg4[1], W_l[:dh], W_l[dh:], W_r,
      b_l.reshape(1, d), fc_w.T, fc_b)

    return h3, y3[..., 0]


# speedup vs baseline: 4.0356x; 1.2227x over previous
import functools

import jax
import jax.numpy as jnp
from jax import lax
from jax.experimental import pallas as pl
from jax.experimental.pallas import tpu as pltpu
from jax.experimental.pallas import tpu_sc as plsc

G = 128
GB = 8
NS = 16
NC = 2
NA = 10240
ZR = 128


def _sc_agg_kernel(seq, n, ep, d):
    dh = d // NC
    nblk = ep // (G * GB)
    bpt = nblk // NS
    rpt = NA // NS
    mesh = plsc.VectorSubcoreMesh(core_axis_name="c", subcore_axis_name="s")

    @functools.partial(
        pl.kernel,
        out_type=[
            jax.ShapeDtypeStruct((NC * seq * NA, dh), jnp.float32),
            jax.ShapeDtypeStruct((NA, 16), jnp.float32),
        ],
        mesh=mesh,
        compiler_params=pltpu.CompilerParams(use_tc_tiling_on_sc=False),
        scratch_types=[
            pltpu.VMEM((2, G, dh), jnp.float32),
            pltpu.VMEM((GB, G), jnp.int32),
            pltpu.VMEM((GB, G), jnp.int32),
            pltpu.VMEM((G, 16), jnp.float32),
            pltpu.VMEM((ZR, dh), jnp.float32),
            pltpu.VMEM((rpt, 16), jnp.float32),
            pltpu.VMEM_SHARED((NA, dh), jnp.float32),
            pltpu.VMEM_SHARED((NA, 16), jnp.float32),
            pltpu.SemaphoreType.DMA,
        ],
    )
    def kern(x_hbm, src_hbm, dst_hbm, out_hbm, deg_hbm,
             rows_v, sidx_v, didx_v, ones_v, zer_v, zdeg_v, agg_s, deg_s, sem):
        c = lax.axis_index("c")
        s = lax.axis_index("s")

        zero16 = jnp.zeros((16,), jnp.float32)
        one16 = jnp.ones((16,), jnp.float32)

        def init_zer(i, _):
            for j in range(dh // 16):
                zer_v[i, pl.ds(j * 16, 16)] = zero16
            return 0
        lax.fori_loop(0, ZR, init_zer, 0)

        def init_zdeg(i, _):
            zdeg_v[i, :] = zero16
            return 0
        lax.fori_loop(0, rpt, init_zdeg, 0)

        def init_ones(i, _):
            ones_v[i, :] = one16
            return 0
        lax.fori_loop(0, G, init_ones, 0)

        r0 = s * rpt
        b0 = s * bpt

        for t in range(seq):
            xt_hbm = x_hbm.at[pl.ds((c * seq + t) * n, n)]

            for j in range(rpt // ZR):
                pltpu.sync_copy(zer_v, agg_s.at[pl.ds(r0 + j * ZR, ZR)])
            if t == 0:
                @pl.when(c == 0)
                def _():
                    pltpu.sync_copy(zdeg_v, deg_s.at[pl.ds(r0, rpt)])
            plsc.subcore_barrier()

            def block_body(b, _):
                pltpu.sync_copy(src_hbm.at[b0 + b], sidx_v)
                pltpu.sync_copy(dst_hbm.at[b0 + b], didx_v)
                cps = [None, None]
                cps[0] = pltpu.async_copy(xt_hbm.at[sidx_v.at[0]],
                                          rows_v.at[0], sem)
                for j in range(GB):
                    if j + 1 < GB:
                        cps[(j + 1) % 2] = pltpu.async_copy(
                            xt_hbm.at[sidx_v.at[j + 1]],
                            rows_v.at[(j + 1) % 2], sem)
                    cps[j % 2].wait()
                    pltpu.sync_copy(rows_v.at[j % 2], agg_s.at[didx_v.at[j]],
                                    add=True)
                    if t == 0:
                        @pl.when(c == 0)
                        def _():
                            pltpu.sync_copy(ones_v, deg_s.at[didx_v.at[j]],
                                            add=True)
                return 0
            lax.fori_loop(0, bpt, block_body, 0)
            plsc.subcore_barrier()

            for j in range(rpt // ZR):
                rr = r0 + j * ZR
                pltpu.sync_copy(agg_s.at[pl.ds(rr, ZR)],
                                out_hbm.at[pl.ds((c * seq + t) * NA + rr, ZR)])
            if t == 0:
                @pl.when(c == 0)
                def _():
                    pltpu.sync_copy(deg_s.at[pl.ds(r0, rpt)],
                                    deg_hbm.at[pl.ds(r0, rpt)])

    return kern


def _tc_body(deg_ref, x_ref, alo_ref, ahi_ref, wll_ref, wlh_ref, wr_ref,
             bl_ref, fcw_ref, fcb_ref, h_ref, y_ref):
    inv = 1.0 / jnp.maximum(deg_ref[:, 0:1], 1.0)
    hp = lax.Precision.HIGHEST
    f32 = jnp.float32
    h = (jnp.dot(alo_ref[0] * inv, wll_ref[...], preferred_element_type=f32,
                 precision=hp)
         + jnp.dot(ahi_ref[0] * inv, wlh_ref[...], preferred_element_type=f32,
                   precision=hp)
         + jnp.dot(x_ref[0], wr_ref[...], preferred_element_type=f32,
                   precision=hp)
         + bl_ref[...])
    h_ref[0] = h
    y_ref[0] = jnp.sum(h * fcw_ref[...], axis=1, keepdims=True) + fcb_ref[0]


def kernel(input, edge_index, W_l, b_l, W_r, fc_w, fc_b):
    seq, n, d = input.shape
    e = edge_index.shape[1]
    dh = d // NC
    epad = G * GB * NS
    ep = ((e + epad - 1) // epad) * epad
    ng = ep // G

    xc = input.reshape(seq * n, NC, dh).transpose(1, 0, 2).reshape(
        NC * seq * n, dh)
    src = edge_index[0]
    dst = edge_index[1]
    if ep != e:
        src = jnp.concatenate([src, jnp.zeros((ep - e,), src.dtype)])
        dst = jnp.concatenate([dst, jnp.full((ep - e,), n, dst.dtype)])
    src_g = src.reshape(ng // GB, GB, G)
    dst_g = dst.reshape(ng // GB, GB, G)

    agg_flat, deg = _sc_agg_kernel(seq, n, ep, d)(xc, src_g, dst_g)
    agg4 = agg_flat.reshape(NC, seq, NA, dh)

    blk = 2000
    h3, y3 = pl.pallas_call(
        _tc_body,
        grid=(seq, n // blk),
        in_specs=[
            pl.BlockSpec((blk, 16), lambda t, b: (b, 0)),
            pl.BlockSpec((1, blk, d), lambda t, b: (t, b, 0)),
            pl.BlockSpec((1, blk, dh), lambda t, b: (t, b, 0)),
            pl.BlockSpec((1, blk, dh), lambda t, b: (t, b, 0)),
            pl.BlockSpec((dh, d), lambda t, b: (0, 0)),
            pl.BlockSpec((dh, d), lambda t, b: (0, 0)),
            pl.BlockSpec((d, d), lambda t, b: (0, 0)),
            pl.BlockSpec((1, d), lambda t, b: (0, 0)),
            pl.BlockSpec((1, d), lambda t, b: (0, 0)),
            pl.BlockSpec(memory_space=pltpu.SMEM),
        ],
        out_specs=[
            pl.BlockSpec((1, blk, d), lambda t, b: (t, b, 0)),
            pl.BlockSpec((1, blk, 1), lambda t, b: (t, b, 0)),
        ],
        out_shape=[
            jax.ShapeDtypeStruct((seq, n, d), jnp.float32),
            jax.ShapeDtypeStruct((seq, n, 1), jnp.float32),
        ],
    )(deg, input, agg4[0], agg4[1], W_l[:dh], W_l[dh:], W_r,
      b_l.reshape(1, d), fc_w.T, fc_b)

    return h3, y3[..., 0]


# speedup vs baseline: 4.1979x; 1.0402x over previous
import functools

import jax
import jax.numpy as jnp
from jax import lax
from jax.experimental import pallas as pl
from jax.experimental.pallas import tpu as pltpu
from jax.experimental.pallas import tpu_sc as plsc

G = 128
GB = 8
NS = 16
NC = 2
NA = 10240
ZR = 128


def _sc_agg_kernel(seq, n, ep, d):
    dh = d // NC
    nblk = ep // (G * GB)
    bpt = nblk // NS
    rpt = NA // NS
    mesh = plsc.VectorSubcoreMesh(core_axis_name="c", subcore_axis_name="s")

    @functools.partial(
        pl.kernel,
        out_type=[
            jax.ShapeDtypeStruct((NC * seq * NA, dh), jnp.float32),
            jax.ShapeDtypeStruct((NA, 16), jnp.float32),
        ],
        mesh=mesh,
        compiler_params=pltpu.CompilerParams(use_tc_tiling_on_sc=False),
        scratch_types=[
            pltpu.VMEM((2, G, dh), jnp.float32),
            pltpu.VMEM((2, GB, G), jnp.int32),
            pltpu.VMEM((2, GB, G), jnp.int32),
            pltpu.VMEM((G, 16), jnp.float32),
            pltpu.VMEM((ZR, dh), jnp.float32),
            pltpu.VMEM((rpt, 16), jnp.float32),
            pltpu.VMEM_SHARED((NA, dh), jnp.float32),
            pltpu.VMEM_SHARED((NA, 16), jnp.float32),
            pltpu.SemaphoreType.DMA,
        ],
    )
    def kern(x_hbm, src_hbm, dst_hbm, out_hbm, deg_hbm,
             rows_v, sidx_v, didx_v, ones_v, zer_v, zdeg_v, agg_s, deg_s, sem):
        c = lax.axis_index("c")
        s = lax.axis_index("s")

        zero16 = jnp.zeros((16,), jnp.float32)
        one16 = jnp.ones((16,), jnp.float32)

        def init_zer(i, _):
            for j in range(dh // 16):
                zer_v[i, pl.ds(j * 16, 16)] = zero16
            return 0
        lax.fori_loop(0, ZR, init_zer, 0)

        def init_zdeg(i, _):
            zdeg_v[i, :] = zero16
            return 0
        lax.fori_loop(0, rpt, init_zdeg, 0)

        def init_ones(i, _):
            ones_v[i, :] = one16
            return 0
        lax.fori_loop(0, G, init_ones, 0)

        r0 = s * rpt
        b0 = s * bpt

        for t in range(seq):
            xt_hbm = x_hbm.at[pl.ds((c * seq + t) * n, n)]

            for j in range(rpt // ZR):
                pltpu.sync_copy(zer_v, agg_s.at[pl.ds(r0 + j * ZR, ZR)])
            if t == 0:
                @pl.when(c == 0)
                def _():
                    pltpu.sync_copy(zdeg_v, deg_s.at[pl.ds(r0, rpt)])
            plsc.subcore_barrier()

            pltpu.sync_copy(src_hbm.at[b0], sidx_v.at[0])
            pltpu.sync_copy(dst_hbm.at[b0], didx_v.at[0])
            pltpu.async_copy(xt_hbm.at[sidx_v.at[0, 0]], rows_v.at[0], sem)

            def pair_body(i, _):
                for k in range(2):
                    cur, nxt = k, (k + 1) % 2
                    b = 2 * i + k

                    @pl.when(b + 1 < bpt)
                    def _():
                        pltpu.sync_copy(src_hbm.at[b0 + b + 1],
                                        sidx_v.at[nxt])
                        pltpu.sync_copy(dst_hbm.at[b0 + b + 1],
                                        didx_v.at[nxt])
                    for j in range(GB):
                        if j + 1 < GB:
                            pltpu.async_copy(xt_hbm.at[sidx_v.at[cur, j + 1]],
                                             rows_v.at[(j + 1) % 2], sem)
                        else:
                            @pl.when(b + 1 < bpt)
                            def _():
                                pltpu.async_copy(
                                    xt_hbm.at[sidx_v.at[nxt, 0]],
                                    rows_v.at[0], sem)
                        pltpu.make_async_copy(xt_hbm.at[sidx_v.at[cur, j]],
                                              rows_v.at[j % 2], sem).wait()
                        pltpu.sync_copy(rows_v.at[j % 2],
                                        agg_s.at[didx_v.at[cur, j]], add=True)
                        if t == 0:
                            @pl.when(c == 0)
                            def _():
                                pltpu.sync_copy(ones_v,
                                                deg_s.at[didx_v.at[cur, j]],
                                                add=True)
                return 0
            lax.fori_loop(0, bpt // 2, pair_body, 0)
            plsc.subcore_barrier()

            for j in range(rpt // ZR):
                rr = r0 + j * ZR
                pltpu.sync_copy(agg_s.at[pl.ds(rr, ZR)],
                                out_hbm.at[pl.ds((c * seq + t) * NA + rr, ZR)])
            if t == 0:
                @pl.when(c == 0)
                def _():
                    pltpu.sync_copy(deg_s.at[pl.ds(r0, rpt)],
                                    deg_hbm.at[pl.ds(r0, rpt)])

    return kern


def _tc_body(deg_ref, x_ref, alo_ref, ahi_ref, wll_ref, wlh_ref, wr_ref,
             bl_ref, fcw_ref, fcb_ref, h_ref, y_ref):
    inv = 1.0 / jnp.maximum(deg_ref[:, 0:1], 1.0)
    hp = lax.Precision.HIGHEST
    f32 = jnp.float32
    h = (jnp.dot(alo_ref[0] * inv, wll_ref[...], preferred_element_type=f32,
                 precision=hp)
         + jnp.dot(ahi_ref[0] * inv, wlh_ref[...], preferred_element_type=f32,
                   precision=hp)
         + jnp.dot(x_ref[0], wr_ref[...], preferred_element_type=f32,
                   precision=hp)
         + bl_ref[...])
    h_ref[0] = h
    y_ref[0] = jnp.sum(h * fcw_ref[...], axis=1, keepdims=True) + fcb_ref[0]


def kernel(input, edge_index, W_l, b_l, W_r, fc_w, fc_b):
    seq, n, d = input.shape
    e = edge_index.shape[1]
    dh = d // NC
    epad = 2 * G * GB * NS
    ep = ((e + epad - 1) // epad) * epad
    ng = ep // G

    xc = input.reshape(seq * n, NC, dh).transpose(1, 0, 2).reshape(
        NC * seq * n, dh)
    src = edge_index[0]
    dst = edge_index[1]
    if ep != e:
        src = jnp.concatenate([src, jnp.zeros((ep - e,), src.dtype)])
        dst = jnp.concatenate([dst, jnp.full((ep - e,), n, dst.dtype)])
    src_g = src.reshape(ng // GB, GB, G)
    dst_g = dst.reshape(ng // GB, GB, G)

    agg_flat, deg = _sc_agg_kernel(seq, n, ep, d)(xc, src_g, dst_g)
    agg4 = agg_flat.reshape(NC, seq, NA, dh)

    blk = 2000
    h3, y3 = pl.pallas_call(
        _tc_body,
        grid=(seq, n // blk),
        in_specs=[
            pl.BlockSpec((blk, 16), lambda t, b: (b, 0)),
            pl.BlockSpec((1, blk, d), lambda t, b: (t, b, 0)),
            pl.BlockSpec((1, blk, dh), lambda t, b: (t, b, 0)),
            pl.BlockSpec((1, blk, dh), lambda t, b: (t, b, 0)),
            pl.BlockSpec((dh, d), lambda t, b: (0, 0)),
            pl.BlockSpec((dh, d), lambda t, b: (0, 0)),
            pl.BlockSpec((d, d), lambda t, b: (0, 0)),
            pl.BlockSpec((1, d), lambda t, b: (0, 0)),
            pl.BlockSpec((1, d), lambda t, b: (0, 0)),
            pl.BlockSpec(memory_space=pltpu.SMEM),
        ],
        out_specs=[
            pl.BlockSpec((1, blk, d), lambda t, b: (t, b, 0)),
            pl.BlockSpec((1, blk, 1), lambda t, b: (t, b, 0)),
        ],
        out_shape=[
            jax.ShapeDtypeStruct((seq, n, d), jnp.float32),
            jax.ShapeDtypeStruct((seq, n, 1), jnp.float32),
        ],
    )(deg, input, agg4[0], agg4[1], W_l[:dh], W_l[dh:], W_r,
      b_l.reshape(1, d), fc_w.T, fc_b)

    return h3, y3[..., 0]


# speedup vs baseline: 4.4647x; 1.0636x over previous
import functools

import jax
import jax.numpy as jnp
from jax import lax
from jax.experimental import pallas as pl
from jax.experimental.pallas import tpu as pltpu
from jax.experimental.pallas import tpu_sc as plsc

G = 128
GB = 8
NS = 16
NC = 2
NA = 10240
ZR = 128


def _sc_agg_kernel(seq, n, ep, d):
    dh = d // NC
    nblk = ep // (G * GB)
    bpt = nblk // NS
    rpt = NA // NS
    mesh = plsc.VectorSubcoreMesh(core_axis_name="c", subcore_axis_name="s")

    @functools.partial(
        pl.kernel,
        out_type=[
            jax.ShapeDtypeStruct((NC * seq * NA, dh), jnp.float32),
            jax.ShapeDtypeStruct((NA, 16), jnp.float32),
        ],
        mesh=mesh,
        compiler_params=pltpu.CompilerParams(use_tc_tiling_on_sc=False),
        scratch_types=[
            pltpu.VMEM((4, G, dh), jnp.float32),
            pltpu.VMEM((2, GB, G), jnp.int32),
            pltpu.VMEM((2, GB, G), jnp.int32),
            pltpu.VMEM((G, 16), jnp.float32),
            pltpu.VMEM((ZR, dh), jnp.float32),
            pltpu.VMEM((rpt, 16), jnp.float32),
            pltpu.VMEM_SHARED((NA, dh), jnp.float32),
            pltpu.VMEM_SHARED((NA, 16), jnp.float32),
            pltpu.SemaphoreType.DMA,
        ],
    )
    def kern(x_hbm, src_hbm, dst_hbm, out_hbm, deg_hbm,
             rows_v, sidx_v, didx_v, ones_v, zer_v, zdeg_v, agg_s, deg_s, sem):
        c = lax.axis_index("c")
        s = lax.axis_index("s")

        zero16 = jnp.zeros((16,), jnp.float32)
        one16 = jnp.ones((16,), jnp.float32)

        def init_zer(i, _):
            for j in range(dh // 16):
                zer_v[i, pl.ds(j * 16, 16)] = zero16
            return 0
        lax.fori_loop(0, ZR, init_zer, 0)

        def init_zdeg(i, _):
            zdeg_v[i, :] = zero16
            return 0
        lax.fori_loop(0, rpt, init_zdeg, 0)

        def init_ones(i, _):
            ones_v[i, :] = one16
            return 0
        lax.fori_loop(0, G, init_ones, 0)

        r0 = s * rpt
        b0 = s * bpt

        for t in range(seq):
            xt_hbm = x_hbm.at[pl.ds((c * seq + t) * n, n)]

            for j in range(rpt // ZR):
                pltpu.sync_copy(zer_v, agg_s.at[pl.ds(r0 + j * ZR, ZR)])
            if t == 0:
                @pl.when(c == 0)
                def _():
                    pltpu.sync_copy(zdeg_v, deg_s.at[pl.ds(r0, rpt)])
            plsc.subcore_barrier()

            LK = 3
            pltpu.sync_copy(src_hbm.at[b0], sidx_v.at[0])
            pltpu.sync_copy(dst_hbm.at[b0], didx_v.at[0])
            for j in range(LK):
                pltpu.async_copy(xt_hbm.at[sidx_v.at[0, j]], rows_v.at[j],
                                 sem)

            def pair_body(i, _):
                for k in range(2):
                    cur, nxt = k, (k + 1) % 2
                    b = 2 * i + k

                    @pl.when(b + 1 < bpt)
                    def _():
                        pltpu.sync_copy(src_hbm.at[b0 + b + 1],
                                        sidx_v.at[nxt])
                        pltpu.sync_copy(dst_hbm.at[b0 + b + 1],
                                        didx_v.at[nxt])
                    for j in range(GB):
                        jf = j + LK
                        if jf < GB:
                            pltpu.async_copy(xt_hbm.at[sidx_v.at[cur, jf]],
                                             rows_v.at[jf % 4], sem)
                        else:
                            @pl.when(b + 1 < bpt)
                            def _():
                                pltpu.async_copy(
                                    xt_hbm.at[sidx_v.at[nxt, jf - GB]],
                                    rows_v.at[jf % 4], sem)
                        pltpu.make_async_copy(xt_hbm.at[sidx_v.at[cur, j]],
                                              rows_v.at[j % 4], sem).wait()
                        pltpu.sync_copy(rows_v.at[j % 4],
                                        agg_s.at[didx_v.at[cur, j]], add=True)
                        if t == 0:
                            @pl.when(c == 0)
                            def _():
                                pltpu.sync_copy(ones_v,
                                                deg_s.at[didx_v.at[cur, j]],
                                                add=True)
                return 0
            lax.fori_loop(0, bpt // 2, pair_body, 0)
            plsc.subcore_barrier()

            for j in range(rpt // ZR):
                rr = r0 + j * ZR
                pltpu.sync_copy(agg_s.at[pl.ds(rr, ZR)],
                                out_hbm.at[pl.ds((c * seq + t) * NA + rr, ZR)])
            if t == 0:
                @pl.when(c == 0)
                def _():
                    pltpu.sync_copy(deg_s.at[pl.ds(r0, rpt)],
                                    deg_hbm.at[pl.ds(r0, rpt)])

    return kern


def _tc_body(deg_ref, x_ref, alo_ref, ahi_ref, wll_ref, wlh_ref, wr_ref,
             bl_ref, fcw_ref, fcb_ref, h_ref, y_ref):
    inv = 1.0 / jnp.maximum(deg_ref[:, 0:1], 1.0)
    hp = lax.Precision.HIGHEST
    f32 = jnp.float32
    h = (jnp.dot(alo_ref[0] * inv, wll_ref[...], preferred_element_type=f32,
                 precision=hp)
         + jnp.dot(ahi_ref[0] * inv, wlh_ref[...], preferred_element_type=f32,
                   precision=hp)
         + jnp.dot(x_ref[0], wr_ref[...], preferred_element_type=f32,
                   precision=hp)
         + bl_ref[...])
    h_ref[0] = h
    y_ref[0] = jnp.sum(h * fcw_ref[...], axis=1, keepdims=True) + fcb_ref[0]


def kernel(input, edge_index, W_l, b_l, W_r, fc_w, fc_b):
    seq, n, d = input.shape
    e = edge_index.shape[1]
    dh = d // NC
    epad = 2 * G * GB * NS
    ep = ((e + epad - 1) // epad) * epad
    ng = ep // G

    xc = input.reshape(seq * n, NC, dh).transpose(1, 0, 2).reshape(
        NC * seq * n, dh)
    src = edge_index[0]
    dst = edge_index[1]
    if ep != e:
        src = jnp.concatenate([src, jnp.zeros((ep - e,), src.dtype)])
        dst = jnp.concatenate([dst, jnp.full((ep - e,), n, dst.dtype)])
    src_g = src.reshape(ng // GB, GB, G)
    dst_g = dst.reshape(ng // GB, GB, G)

    agg_flat, deg = _sc_agg_kernel(seq, n, ep, d)(xc, src_g, dst_g)
    agg4 = agg_flat.reshape(NC, seq, NA, dh)

    blk = 2000
    h3, y3 = pl.pallas_call(
        _tc_body,
        grid=(seq, n // blk),
        in_specs=[
            pl.BlockSpec((blk, 16), lambda t, b: (b, 0)),
            pl.BlockSpec((1, blk, d), lambda t, b: (t, b, 0)),
            pl.BlockSpec((1, blk, dh), lambda t, b: (t, b, 0)),
            pl.BlockSpec((1, blk, dh), lambda t, b: (t, b, 0)),
            pl.BlockSpec((dh, d), lambda t, b: (0, 0)),
            pl.BlockSpec((dh, d), lambda t, b: (0, 0)),
            pl.BlockSpec((d, d), lambda t, b: (0, 0)),
            pl.BlockSpec((1, d), lambda t, b: (0, 0)),
            pl.BlockSpec((1, d), lambda t, b: (0, 0)),
            pl.BlockSpec(memory_space=pltpu.SMEM),
        ],
        out_specs=[
            pl.BlockSpec((1, blk, d), lambda t, b: (t, b, 0)),
            pl.BlockSpec((1, blk, 1), lambda t, b: (t, b, 0)),
        ],
        out_shape=[
            jax.ShapeDtypeStruct((seq, n, d), jnp.float32),
            jax.ShapeDtypeStruct((seq, n, 1), jnp.float32),
        ],
    )(deg, input, agg4[0], agg4[1], W_l[:dh], W_l[dh:], W_r,
      b_l.reshape(1, d), fc_w.T, fc_b)

    return h3, y3[..., 0]


# speedup vs baseline: 4.6039x; 1.0312x over previous
import functools

import jax
import jax.numpy as jnp
from jax import lax
from jax.experimental import pallas as pl
from jax.experimental.pallas import tpu as pltpu
from jax.experimental.pallas import tpu_sc as plsc

G = 128
GB = 8
NS = 16
NC = 2
NA = 10240
ZR = 64


def _sc_agg_kernel(seq, n, ep, d):
    dh = d // NC
    nblk = ep // (G * GB)
    bpt = nblk // NS
    rpt = NA // NS
    mesh = plsc.VectorSubcoreMesh(core_axis_name="c", subcore_axis_name="s")

    @functools.partial(
        pl.kernel,
        out_type=[
            jax.ShapeDtypeStruct((NC * seq * NA, dh), jnp.float32),
            jax.ShapeDtypeStruct((NA, 16), jnp.float32),
        ],
        mesh=mesh,
        compiler_params=pltpu.CompilerParams(use_tc_tiling_on_sc=False),
        scratch_types=[
            pltpu.VMEM((8, G, dh), jnp.float32),
            pltpu.VMEM((2, GB, G), jnp.int32),
            pltpu.VMEM((2, GB, G), jnp.int32),
            pltpu.VMEM((G, 16), jnp.float32),
            pltpu.VMEM((ZR, dh), jnp.float32),
            pltpu.VMEM((ZR, 16), jnp.float32),
            pltpu.VMEM_SHARED((NA, dh), jnp.float32),
            pltpu.VMEM_SHARED((NA, 16), jnp.float32),
            pltpu.SemaphoreType.DMA,
            pltpu.SemaphoreType.DMA,
        ],
    )
    def kern(x_hbm, src_hbm, dst_hbm, out_hbm, deg_hbm,
             rows_v, sidx_v, didx_v, ones_v, zer_v, zdeg_v, agg_s, deg_s,
             sem, sem2):
        c = lax.axis_index("c")
        s = lax.axis_index("s")

        zero16 = jnp.zeros((16,), jnp.float32)
        one16 = jnp.ones((16,), jnp.float32)

        def init_zer(i, _):
            for j in range(dh // 16):
                zer_v[i, pl.ds(j * 16, 16)] = zero16
            return 0
        lax.fori_loop(0, ZR, init_zer, 0)

        def init_zdeg(i, _):
            zdeg_v[i, :] = zero16
            return 0
        lax.fori_loop(0, ZR, init_zdeg, 0)

        def init_ones(i, _):
            ones_v[i, :] = one16
            return 0
        lax.fori_loop(0, G, init_ones, 0)

        r0 = s * rpt
        b0 = s * bpt

        for t in range(seq):
            xt_hbm = x_hbm.at[pl.ds((c * seq + t) * n, n)]

            for j in range(rpt // ZR):
                pltpu.sync_copy(zer_v, agg_s.at[pl.ds(r0 + j * ZR, ZR)])
            if t == 0:
                @pl.when(c == 0)
                def _():
                    for j in range(rpt // ZR):
                        pltpu.sync_copy(zdeg_v,
                                        deg_s.at[pl.ds(r0 + j * ZR, ZR)])
            plsc.subcore_barrier()

            LK = 5
            pltpu.sync_copy(src_hbm.at[b0], sidx_v.at[0])
            pltpu.sync_copy(dst_hbm.at[b0], didx_v.at[0])
            for j in range(LK):
                pltpu.async_copy(xt_hbm.at[sidx_v.at[0, j]], rows_v.at[j],
                                 sem)

            def wait_scat():
                pltpu.make_async_copy(rows_v.at[0],
                                      agg_s.at[didx_v.at[0, 0]],
                                      sem2).wait()

            def pair_body(i, _):
                for k in range(2):
                    cur, nxt = k, (k + 1) % 2
                    b = 2 * i + k

                    @pl.when(b + 1 < bpt)
                    def _():
                        pltpu.sync_copy(src_hbm.at[b0 + b + 1],
                                        sidx_v.at[nxt])
                        pltpu.sync_copy(dst_hbm.at[b0 + b + 1],
                                        didx_v.at[nxt])
                    for j in range(GB):
                        if j - (GB - LK) >= 0:
                            wait_scat()
                        elif k == 1:
                            wait_scat()
                        else:
                            @pl.when(i >= 1)
                            def _():
                                wait_scat()
                        jf = j + LK
                        if jf < GB:
                            pltpu.async_copy(xt_hbm.at[sidx_v.at[cur, jf]],
                                             rows_v.at[jf % 8], sem)
                        else:
                            @pl.when(b + 1 < bpt)
                            def _():
                                pltpu.async_copy(
                                    xt_hbm.at[sidx_v.at[nxt, jf - GB]],
                                    rows_v.at[jf % 8], sem)
                        pltpu.make_async_copy(xt_hbm.at[sidx_v.at[cur, j]],
                                              rows_v.at[j % 8], sem).wait()
                        pltpu.async_copy(rows_v.at[j % 8],
                                         agg_s.at[didx_v.at[cur, j]], sem2,
                                         add=True)
                        if t == 0:
                            @pl.when(c == 0)
                            def _():
                                pltpu.sync_copy(ones_v,
                                                deg_s.at[didx_v.at[cur, j]],
                                                add=True)
                return 0
            lax.fori_loop(0, bpt // 2, pair_body, 0)
            for _ in range(GB - LK):
                wait_scat()
            plsc.subcore_barrier()

            pltpu.sync_copy(agg_s.at[pl.ds(r0, rpt)],
                            out_hbm.at[pl.ds((c * seq + t) * NA + r0, rpt)])
            if t == 0:
                @pl.when(c == 0)
                def _():
                    pltpu.sync_copy(deg_s.at[pl.ds(r0, rpt)],
                                    deg_hbm.at[pl.ds(r0, rpt)])

    return kern


def _tc_body(deg_ref, x_ref, alo_ref, ahi_ref, wll_ref, wlh_ref, wr_ref,
             bl_ref, fcw_ref, fcb_ref, h_ref, y_ref):
    inv = 1.0 / jnp.maximum(deg_ref[:, 0:1], 1.0)
    hp = lax.Precision.HIGHEST
    f32 = jnp.float32
    h = (jnp.dot(alo_ref[0] * inv, wll_ref[...], preferred_element_type=f32,
                 precision=hp)
         + jnp.dot(ahi_ref[0] * inv, wlh_ref[...], preferred_element_type=f32,
                   precision=hp)
         + jnp.dot(x_ref[0], wr_ref[...], preferred_element_type=f32,
                   precision=hp)
         + bl_ref[...])
    h_ref[0] = h
    y_ref[0] = jnp.sum(h * fcw_ref[...], axis=1, keepdims=True) + fcb_ref[0]


def kernel(input, edge_index, W_l, b_l, W_r, fc_w, fc_b):
    seq, n, d = input.shape
    e = edge_index.shape[1]
    dh = d // NC
    epad = 2 * G * GB * NS
    ep = ((e + epad - 1) // epad) * epad
    ng = ep // G

    xc = input.reshape(seq * n, NC, dh).transpose(1, 0, 2).reshape(
        NC * seq * n, dh)
    src = edge_index[0]
    dst = edge_index[1]
    if ep != e:
        src = jnp.concatenate([src, jnp.zeros((ep - e,), src.dtype)])
        dst = jnp.concatenate([dst, jnp.full((ep - e,), n, dst.dtype)])
    src_g = src.reshape(ng // GB, GB, G)
    dst_g = dst.reshape(ng // GB, GB, G)

    agg_flat, deg = _sc_agg_kernel(seq, n, ep, d)(xc, src_g, dst_g)
    agg4 = agg_flat.reshape(NC, seq, NA, dh)

    blk = 2000
    h3, y3 = pl.pallas_call(
        _tc_body,
        grid=(seq, n // blk),
        in_specs=[
            pl.BlockSpec((blk, 16), lambda t, b: (b, 0)),
            pl.BlockSpec((1, blk, d), lambda t, b: (t, b, 0)),
            pl.BlockSpec((1, blk, dh), lambda t, b: (t, b, 0)),
            pl.BlockSpec((1, blk, dh), lambda t, b: (t, b, 0)),
            pl.BlockSpec((dh, d), lambda t, b: (0, 0)),
            pl.BlockSpec((dh, d), lambda t, b: (0, 0)),
            pl.BlockSpec((d, d), lambda t, b: (0, 0)),
            pl.BlockSpec((1, d), lambda t, b: (0, 0)),
            pl.BlockSpec((1, d), lambda t, b: (0, 0)),
            pl.BlockSpec(memory_space=pltpu.SMEM),
        ],
        out_specs=[
            pl.BlockSpec((1, blk, d), lambda t, b: (t, b, 0)),
            pl.BlockSpec((1, blk, 1), lambda t, b: (t, b, 0)),
        ],
        out_shape=[
            jax.ShapeDtypeStruct((seq, n, d), jnp.float32),
            jax.ShapeDtypeStruct((seq, n, 1), jnp.float32),
        ],
    )(deg, input, agg4[0], agg4[1], W_l[:dh], W_l[dh:], W_r,
      b_l.reshape(1, d), fc_w.T, fc_b)

    return h3, y3[..., 0]
